# baseline (speedup 1.0000x reference)
import jax
import jax.numpy as jnp
from jax import lax
from jax.experimental import pallas as pl
from jax.experimental.pallas import tpu as pltpu
from jax.experimental.pallas import tpu_sc as plsc

B, H, W, C = 4, 112, 112, 96
OH, OW = 2 * H, 2 * W
M = OH * OW * C

NSC, NTILE = 2, 16
ROWS_PT = H // NTILE
CHW = 16
NCHROW = W // CHW
NCH = ROWS_PT * NCHROW
CHVAL = CHW * C
NBUF = 3

NWIN = 4
WROWS = OH // NWIN
ROWW = OW * C
WSZ = WROWS * ROWW
DUMSZ = 2048
ZCH = 4096
XH = OW // 2

_THIRD = 1.0 / 3.0


def _sc_body(upd_hbm, msk_hbm, out_hbm, bufs, zero_v, out1d, out_s, win,
             sin, ssc):
    cid = lax.axis_index("c")
    sid = lax.axis_index("s")
    lane = lax.iota(jnp.int32, 16)

    def zfill(i, _):
        zero_v[pl.ds(i * 16, 16)] = jnp.zeros((16,), jnp.float32)
        return 0

    lax.fori_loop(0, ZCH // 16, zfill, 0)

    def one_pass(p, _):
        b, w = p // NWIN, p % NWIN
        w0 = w * WSZ

        @pl.when((p % 2) == cid)
        def _run():
            row0 = sid * ROWS_PT

            def start_in(r, x0, q):
                msk_v, upd_v, _, _ = bufs[q]
                x0 = pl.multiple_of(x0, CHW)
                pltpu.async_copy(
                    msk_hbm.at[b, r, pl.ds(x0, CHW), :], msk_v, sin[q])
                pltpu.async_copy(
                    upd_hbm.at[b, r, pl.ds(x0, CHW), :], upd_v, sin[q])

            def wait_in(q):
                msk_v, upd_v, _, _ = bufs[q]
                pltpu.make_async_copy(
                    msk_hbm.at[0, 0, pl.ds(0, CHW), :], msk_v, sin[q]).wait()
                pltpu.make_async_copy(
                    upd_hbm.at[0, 0, pl.ds(0, CHW), :], upd_v, sin[q]).wait()

            def wait_sc(q):
                _, _, idx_v, cupd_v = bufs[q]
                pltpu.make_async_copy(cupd_v, win.at[idx_v], ssc[q]).wait()

            start_in(row0, 0, 0)

            def zbody(k, _):
                pltpu.sync_copy(zero_v,
                                win.at[pl.ds(sid * (WSZ // NTILE) + k * ZCH,
                                             ZCH)])
                return 0

            nz = (WSZ // NTILE) // ZCH
            zrem = WSZ // NTILE - nz * ZCH
            lax.fori_loop(0, nz, zbody, 0)
            pltpu.sync_copy(zero_v.at[pl.ds(0, zrem)],
                            win.at[pl.ds(sid * (WSZ // NTILE) + nz * ZCH,
                                         zrem)])
            plsc.subcore_barrier()

            dumbase = WSZ + sid * DUMSZ

            NGRP = (NCH + NBUF - 1) // NBUF

            def group(g, carry):
                r, x0 = carry
                for i in range(NBUF):
                    j = g * NBUF + i
                    q, qn = i % NBUF, (i + 1) % NBUF

                    @pl.when(j < NCH)
                    def _chunk(j=j, q=q, qn=qn, r=r, x0=x0):
                        @pl.when(j + 1 < NCH)
                        def _pre():
                            @pl.when(j + 1 - NBUF >= 0)
                            def _wsc():
                                wait_sc(qn)

                            start_in(r, x0, qn)

                        wait_in(q)
                        msk_v, upd_v, idx_v, cupd_v = bufs[q]

                        def vbody(x_, _):
                            for u in range(C // 16):
                                m = msk_v[x_, pl.ds(u * 16, 16)]
                                q32 = ((m >> 5).astype(jnp.float32) * _THIRD
                                       + 0.1).astype(jnp.int32)
                                rel = q32 * 96 + (lane + 16 * u) - w0
                                dummy = dumbase + u * 256 + x_ * 16 + lane
                                inw = (m >= w0) & (m < w0 + WSZ)
                                off = x_ * C + u * 16
                                idx_v[pl.ds(off, 16)] = jnp.where(
                                    inw, rel, dummy)
                                cupd_v[pl.ds(off, 16)] = (
                                    upd_v[x_, pl.ds(u * 16, 16)])
                            return 0

                        lax.fori_loop(0, CHW, vbody, 0)
                        pltpu.async_copy(cupd_v, win.at[idx_v], ssc[q],
                                         add=True)

                    x0n = x0 + CHW
                    wrap = x0n >= W
                    r = jnp.where(wrap, r + 1, r)
                    x0 = jnp.where(wrap, 0, x0n)
                return r, x0

            lax.fori_loop(0, NGRP, group, (row0, CHW))

            for j in range(NCH - NBUF, NCH):
                wait_sc(j % NBUF)
            plsc.subcore_barrier()

            def rbody(x_, _):
                for u in range(C // 16):
                    out_s[x_, pl.ds(u * 16, 16)] = (
                        out1d[pl.ds(x_ * C + u * 16, 16)])
                return 0

            for k in range(4):
                r = sid + k * NTILE

                @pl.when(r < WROWS)
                def _row(r=r):
                    y = w * WROWS + r
                    for xh in range(2):
                        pltpu.sync_copy(
                            win.at[pl.ds(r * ROWW + xh * (XH * C), XH * C)],
                            out1d)
                        lax.fori_loop(0, XH, rbody, 0)
                        pltpu.sync_copy(
                            out_s, out_hbm.at[b, y, pl.ds(xh * XH, XH), :])

            plsc.subcore_barrier()

        return 0

    lax.fori_loop(0, B * NWIN, one_pass, 0)


_unpool_sc = pl.kernel(
    _sc_body,
    out_type=jax.ShapeDtypeStruct((B, OH, OW, C), jnp.float32),
    mesh=plsc.VectorSubcoreMesh(core_axis_name="c", subcore_axis_name="s"),
    compiler_params=pltpu.CompilerParams(use_tc_tiling_on_sc=True),
    scratch_types=[
        [(pltpu.VMEM((CHW, C), jnp.int32),
          pltpu.VMEM((CHW, C), jnp.float32),
          pltpu.VMEM((CHVAL,), jnp.int32),
          pltpu.VMEM((CHVAL,), jnp.float32))
         for _ in range(NBUF)],
        pltpu.VMEM((ZCH,), jnp.float32),
        pltpu.VMEM((XH * C,), jnp.float32),
        pltpu.VMEM((XH, C), jnp.float32),
        pltpu.VMEM_SHARED((WSZ + NTILE * DUMSZ,), jnp.float32),
        [pltpu.SemaphoreType.DMA for _ in range(NBUF)],
        [pltpu.SemaphoreType.DMA for _ in range(NBUF)],
    ],
)


@jax.jit
def kernel(updates, mask):
    return _unpool_sc(updates, mask.astype(jnp.int32))


# speedup vs baseline: 1.7609x; 1.7609x over previous
import jax
import jax.numpy as jnp
from jax import lax
from jax.experimental import pallas as pl
from jax.experimental.pallas import tpu as pltpu
from jax.experimental.pallas import tpu_sc as plsc

B, H, W, C = 4, 112, 112, 96
OH, OW = 2 * H, 2 * W
N = H * W * C
M = OH * OW * C

NSC, NTILE = 2, 16
NPER = N // NTILE
CH = 4704
NCH = NPER // CH
UNROLL = 6
VITER = CH // (16 * UNROLL)
NBUF = 3

NWIN = 4
WMAX = M // NWIN
OUTCH = WMAX // NTILE
DUMSZ = 1024
ZCH = 2048
OCH = 4096
NOCH = OUTCH // OCH
OREM = OUTCH - NOCH * OCH

_THIRD = 1.0 / 3.0


def _sc_body(upd_hbm, msk_hbm, out_hbm, bufs, zero_v, ostg, win, sin, ssc,
             sout, szo):
    cid = lax.axis_index("c")
    sid = lax.axis_index("s")
    lane = lax.iota(jnp.int32, 16)

    def zfill(i, _):
        zero_v[pl.ds(i * 16, 16)] = jnp.zeros((16,), jnp.float32)
        return 0

    lax.fori_loop(0, ZCH // 16, zfill, 0)

    def one_pass(p, _):
        b, w = p // NWIN, p % NWIN
        w0 = w * WMAX

        @pl.when((p % 2) == cid)
        def _run():
            base0 = b * N + sid * NPER

            def start_in(j, q):
                msk_v, upd_v, _ = bufs[q]
                pltpu.async_copy(
                    msk_hbm.at[pl.ds(base0 + j * CH, CH)], msk_v, sin[q])
                pltpu.async_copy(
                    upd_hbm.at[pl.ds(base0 + j * CH, CH)], upd_v, sin[q])

            def wait_in(q):
                msk_v, upd_v, _ = bufs[q]
                pltpu.make_async_copy(
                    msk_hbm.at[pl.ds(0, CH)], msk_v, sin[q]).wait()
                pltpu.make_async_copy(
                    upd_hbm.at[pl.ds(0, CH)], upd_v, sin[q]).wait()

            def wait_sc(q):
                _, upd_v, idx_v = bufs[q]
                pltpu.make_async_copy(upd_v, win.at[idx_v], ssc[q]).wait()

            start_in(0, 0)

            @pl.when(p < 2)
            def _prezero():
                def zbody(k, _):
                    pltpu.sync_copy(
                        zero_v, win.at[pl.ds(sid * OUTCH + k * ZCH, ZCH)])
                    return 0

                nzp = OUTCH // ZCH
                lax.fori_loop(0, nzp, zbody, 0)
                pltpu.sync_copy(
                    zero_v.at[pl.ds(0, OUTCH - nzp * ZCH)],
                    win.at[pl.ds(sid * OUTCH + nzp * ZCH,
                                 OUTCH - nzp * ZCH)])

            plsc.subcore_barrier()

            dumbase = WMAX + sid * DUMSZ

            for j in range(NCH):
                q, qn = j % NBUF, (j + 1) % NBUF
                if j + 1 < NCH:
                    if j + 1 - NBUF >= 0:
                        wait_sc(qn)
                    start_in(j + 1, qn)
                wait_in(q)
                msk_v, upd_v, idx_v = bufs[q]

                def vbody(i, _, msk_v=msk_v, idx_v=idx_v):
                    for u in range(UNROLL):
                        off = i * (16 * UNROLL) + u * 16
                        m = msk_v[pl.ds(off, 16)]
                        q32 = ((m >> 5).astype(jnp.float32) * _THIRD
                               + 0.1).astype(jnp.int32)
                        rel = q32 * 96 + (lane + 16 * u) - w0
                        dummy = dumbase + i * 16 + lane
                        inw = (m >= w0) & (m < w0 + WMAX)
                        idx_v[pl.ds(off, 16)] = jnp.where(inw, rel, dummy)
                    return 0

                lax.fori_loop(0, VITER, vbody, 0)
                pltpu.async_copy(upd_v, win.at[idx_v], ssc[q], add=True)

            for j in range(NCH - NBUF, NCH):
                wait_sc(j % NBUF)
            plsc.subcore_barrier()

            hbase = b * M + w0 + sid * OUTCH

            for k in range(NOCH + 1):
                o = k % 2
                sz = OCH if k < NOCH else OREM
                src = sid * OUTCH + k * OCH
                if k >= 2:
                    pltpu.make_async_copy(
                        ostg[o], out_hbm.at[pl.ds(0, OCH)], sout[o]).wait()
                pltpu.sync_copy(win.at[pl.ds(src, sz)],
                                ostg[o].at[pl.ds(0, sz)])
                pltpu.async_copy(ostg[o].at[pl.ds(0, sz)],
                                 out_hbm.at[pl.ds(hbase + k * OCH, sz)],
                                 sout[o])
                pltpu.async_copy(zero_v.at[pl.ds(0, min(sz, ZCH))],
                                 win.at[pl.ds(src, min(sz, ZCH))], szo)
                if sz > ZCH:
                    pltpu.async_copy(zero_v,
                                     win.at[pl.ds(src + ZCH, ZCH)], szo)

            pltpu.make_async_copy(
                ostg[NOCH % 2], out_hbm.at[pl.ds(0, OREM)],
                sout[NOCH % 2]).wait()
            pltpu.make_async_copy(
                ostg[(NOCH - 1) % 2], out_hbm.at[pl.ds(0, OCH)],
                sout[(NOCH - 1) % 2]).wait()

            def zdrain(k, _):
                pltpu.make_async_copy(
                    zero_v, win.at[pl.ds(0, ZCH)], szo).wait()
                return 0

            lax.fori_loop(0, 2 * NOCH, zdrain, 0)
            pltpu.make_async_copy(
                zero_v.at[pl.ds(0, OREM)], win.at[pl.ds(0, OREM)], szo).wait()
            plsc.subcore_barrier()

        return 0

    lax.fori_loop(0, B * NWIN, one_pass, 0)


_unpool_sc = pl.kernel(
    _sc_body,
    out_type=jax.ShapeDtypeStruct((B * M,), jnp.float32),
    mesh=plsc.VectorSubcoreMesh(core_axis_name="c", subcore_axis_name="s"),
    scratch_types=[
        [(pltpu.VMEM((CH,), jnp.int32),
          pltpu.VMEM((CH,), jnp.float32),
          pltpu.VMEM((CH,), jnp.int32))
         for _ in range(NBUF)],
        pltpu.VMEM((ZCH,), jnp.float32),
        [pltpu.VMEM((OCH,), jnp.float32) for _ in range(2)],
        pltpu.VMEM_SHARED((WMAX + NTILE * DUMSZ,), jnp.float32),
        [pltpu.SemaphoreType.DMA for _ in range(NBUF)],
        [pltpu.SemaphoreType.DMA for _ in range(NBUF)],
        [pltpu.SemaphoreType.DMA for _ in range(2)],
        pltpu.SemaphoreType.DMA,
    ],
)


@jax.jit
def kernel(updates, mask):
    upd = updates.reshape(B * N)
    msk = mask.astype(jnp.int32).reshape(B * N)
    out = _unpool_sc(upd, msk)
    return out.reshape(B, OH, OW, C)
